# baseline (speedup 1.0000x reference)
import jax
import jax.numpy as jnp
from jax.experimental import pallas as pl

_EPS = 1e-08


def _block(n: int, target: int) -> int:
    for b in range(min(n, target), 7, -1):
        if n % b == 0 and b % 8 == 0:
            return b
    return n


def _prep_body(adj_ref, x_ref, w_ref, d_ref, g_ref):
    deg = jnp.sum(adj_ref[...], axis=1, keepdims=True) + 1.0
    dis = jax.lax.rsqrt(jnp.maximum(deg, _EPS))
    d_ref[...] = dis
    h = jax.lax.dot_general(
        x_ref[...], w_ref[...], (((1,), (1,)), ((), ())),
        preferred_element_type=jnp.float32)
    g_ref[...] = dis * h


def _mm_body(adj_ref, g_ref, gi_ref, d_ref, o_ref):
    y = jnp.dot(adj_ref[...], g_ref[...], preferred_element_type=jnp.float32)
    o_ref[...] = jnp.maximum(d_ref[...] * (y + gi_ref[...]), 0.0)


def kernel(x, adj, W):
    n, f_in = x.shape
    f_out = W.shape[0]

    bm = _block(n, 400)
    ni = n // bm

    d, g = pl.pallas_call(
        _prep_body,
        grid=(ni,),
        in_specs=[
            pl.BlockSpec((bm, n), lambda i: (i, 0)),
            pl.BlockSpec((bm, f_in), lambda i: (i, 0)),
            pl.BlockSpec((f_out, f_in), lambda i: (0, 0)),
        ],
        out_specs=[
            pl.BlockSpec((bm, 1), lambda i: (i, 0)),
            pl.BlockSpec((bm, f_out), lambda i: (i, 0)),
        ],
        out_shape=[
            jax.ShapeDtypeStruct((n, 1), jnp.float32),
            jax.ShapeDtypeStruct((n, f_out), jnp.float32),
        ],
    )(adj, x, W)

    out = pl.pallas_call(
        _mm_body,
        grid=(ni,),
        in_specs=[
            pl.BlockSpec((bm, n), lambda i: (i, 0)),
            pl.BlockSpec((n, f_out), lambda i: (0, 0)),
            pl.BlockSpec((bm, f_out), lambda i: (i, 0)),
            pl.BlockSpec((bm, 1), lambda i: (i, 0)),
        ],
        out_specs=pl.BlockSpec((bm, f_out), lambda i: (i, 0)),
        out_shape=jax.ShapeDtypeStruct((n, f_out), jnp.float32),
    )(adj, g, g, d)

    return out
